# baseline (speedup 1.0000x reference)
import functools

import jax
import jax.numpy as jnp
from jax import lax
from jax.experimental import pallas as pl
from jax.experimental.pallas import tpu as pltpu
from jax.experimental.pallas import tpu_sc as plsc

N_NODES = 10000
N_EDGES = 160000
NP = 10240
DUMMY = N_NODES
NC, NS = 2, 16
NW = NC * NS
EB = 64
EBD = 128
NBD = 40
E_PAD = EBD * NBD * NW
RPS = NP // NS
NB0, NB1 = 81, 81
NBUF = 3
EPG = NC * NS * NB0 * EB


_MESH = plsc.VectorSubcoreMesh(core_axis_name="c", subcore_axis_name="s")


def _make_sc_agg(n_chunks, dc):
    n = n_chunks

    def body(*refs):
        src4, dst4, zeros = refs[:3]
        tables = refs[3:3 + n]
        outs = refs[3 + n:3 + n + n]
        srcv, dstv = refs[3 + n + n:5 + n + n]
        gbufs = refs[5 + n + n:5 + n + n + NBUF]
        acc = refs[5 + n + n + NBUF]
        sems = refs[6 + n + n + NBUF:6 + n + n + 2 * NBUF]

        c = lax.axis_index("c")
        s = lax.axis_index("s")
        w = c * NS + s
        nb = lax.select(c == 0, NB0, NB1)

        pltpu.sync_copy(src4.at[w], srcv)
        pltpu.sync_copy(dst4.at[w], dstv)

        for i in range(n):
            pltpu.sync_copy(zeros.at[pl.ds(s * RPS, RPS)], acc.at[pl.ds(s * RPS, RPS)])
            plsc.subcore_barrier()

            tbl = tables[i]
            for b in range(NBUF - 1):
                pltpu.async_copy(tbl.at[srcv.at[b]], gbufs[b], sems[b])

            def edge_body(i4, carry):
                for b in range(NBUF):
                    k = i4 * NBUF + b

                    @pl.when(k < nb)
                    def _():
                        pltpu.make_async_copy(tbl.at[srcv.at[k]],
                                              gbufs[b], sems[b]).wait()
                        pltpu.sync_copy(gbufs[b], acc.at[dstv.at[k]], add=True)

                        @pl.when(k + NBUF - 1 < nb)
                        def _():
                            kn = k + NBUF - 1
                            bn = (b + NBUF - 1) % NBUF
                            pltpu.async_copy(tbl.at[srcv.at[kn]],
                                             gbufs[bn], sems[bn])

                return carry

            lax.fori_loop(0, NB0 // NBUF, edge_body, 0)
            plsc.subcore_barrier()
            pltpu.sync_copy(acc.at[pl.ds(s * RPS, RPS)],
                            outs[i].at[c, pl.ds(s * RPS, RPS)])

    return pl.kernel(
        body,
        out_type=[jax.ShapeDtypeStruct((NC, NP, dc), jnp.float32)] * n,
        mesh=_MESH,
        scratch_types=[
            pltpu.VMEM((NB0, EB), jnp.int32),
            pltpu.VMEM((NB0, EB), jnp.int32),
            *[pltpu.VMEM((EB, dc), jnp.float32) for _ in range(NBUF)],
            pltpu.VMEM_SHARED((NP, dc), jnp.float32),
            *[pltpu.SemaphoreType.DMA for _ in range(NBUF)],
        ],
    )


def _sc_deg(dst3, z128, ones128):
    def body(dst3, z128, ones128, out, dstv, onesv, acc):
        c = lax.axis_index("c")
        s = lax.axis_index("s")
        w = c * NS + s
        pltpu.sync_copy(dst3.at[w], dstv)
        pltpu.sync_copy(ones128, onesv)
        pltpu.sync_copy(z128.at[pl.ds(s * RPS, RPS)], acc.at[pl.ds(s * RPS, RPS)])
        plsc.subcore_barrier()

        def edge_body(k, carry):
            pltpu.sync_copy(onesv, acc.at[dstv.at[k]], add=True)
            return carry

        lax.fori_loop(0, NBD, edge_body, 0)
        plsc.subcore_barrier()
        pltpu.sync_copy(acc.at[pl.ds(s * RPS, RPS)], out.at[c, pl.ds(s * RPS, RPS)])

    return pl.kernel(
        body,
        out_type=jax.ShapeDtypeStruct((NC, NP, 128), jnp.float32),
        mesh=_MESH,
        scratch_types=[
            pltpu.VMEM((NBD, EBD), jnp.int32),
            pltpu.VMEM((EBD, 128), jnp.float32),
            pltpu.VMEM_SHARED((NP, 128), jnp.float32),
        ],
    )(dst3, z128, ones128)


R = 400
G = N_NODES // R


def _l1_body(px, pd, x, w1l, w1r, b1, o0, o1, o2, o3, invd):
    deg = (pd[0] + pd[1])[:, 0:16]
    inv = 1.0 / jnp.maximum(deg, 1.0)
    agg = (px[0] + px[1]) * inv[:, 0:1]
    z = (jnp.dot(agg, w1l[...], preferred_element_type=jnp.float32)
         + jnp.dot(x[...], w1r[...], preferred_element_type=jnp.float32)
         + b1[0][None, :])
    h = jnp.maximum(z, 0.0)
    o0[...] = h[:, 0:128]
    o1[...] = h[:, 128:256]
    o2[...] = h[:, 256:384]
    o3[...] = h[:, 384:512]
    invd[...] = inv


def _l2_body(p0, p1, p2, p3, h0, h1, h2c, h3, invd, w2l, w2r, b2, w3lp, w3rp, b3p,
             p_out, pr_out):
    inv1 = invd[...][:, 0:1]
    hs = (h0, h1, h2c, h3)
    ps = (p0, p1, p2, p3)
    z = b2[0][None, :]
    for c in range(4):
        aggc = (ps[c][0] + ps[c][1]) * inv1
        z = z + jnp.dot(aggc, w2l[c], preferred_element_type=jnp.float32)
        z = z + jnp.dot(hs[c][...], w2r[c], preferred_element_type=jnp.float32)
    h2 = jnp.maximum(z, 0.0)
    p = jnp.zeros((R, 128), jnp.float32)
    pr = b3p[0][None, :]
    for c in range(8):
        blk = h2[:, c * 128:(c + 1) * 128]
        p = p + jnp.dot(blk, w3lp[c], preferred_element_type=jnp.float32)
        pr = pr + jnp.dot(blk, w3rp[c], preferred_element_type=jnp.float32)
    p_out[...] = p
    pr_out[...] = pr


def _l3_body(pp, pr, invd, out):
    agg = (pp[0] + pp[1])[:, 0:16] * invd[...]
    out[...] = jnp.maximum(agg + pr[...], 0.0)


def _row_spec(shape_prefix, r, dc):
    nd = len(shape_prefix)
    return pl.BlockSpec(shape_prefix + (r, dc),
                        lambda i, nd=nd: (0,) * nd + (i, 0))


def _full_spec(shape):
    return pl.BlockSpec(shape, lambda i: (0,) * len(shape))


def kernel(x, edge_index, batch, W1_l, W1_r, b1, W2_l, W2_r, b2, W3_l, W3_r, b3):
    src = edge_index[0]
    dst = edge_index[1]
    dst3 = jnp.concatenate(
        [dst, jnp.full((E_PAD - N_EDGES,), DUMMY, jnp.int32)]).reshape(NW, NBD, EBD)

    def _skew(e, fill):
        ep = jnp.concatenate([e, jnp.full((EPG - N_EDGES,), fill, jnp.int32)])
        n0 = NS * NB0 * EB
        e0 = ep[:n0].reshape(NS, NB0, EB)
        e1 = ep[n0:].reshape(NS, NB1, EB)
        e1 = jnp.pad(e1, ((0, 0), (0, NB0 - NB1), (0, 0)))
        return jnp.concatenate([e0, e1])

    src4 = _skew(src, 0)
    dst4 = _skew(dst, DUMMY)
    z128 = jnp.zeros((NP, 128), jnp.float32)
    ones128 = jnp.ones((EBD, 128), jnp.float32)

    (px,) = _make_sc_agg(1, 128)(src4, dst4, z128, x)
    pdeg = _sc_deg(dst3, z128, ones128)

    w1l = W1_l
    w1r = W1_r
    b1r = b1[None, :]
    h1c = pl.pallas_call(
        _l1_body,
        grid=(G,),
        in_specs=[
            _row_spec((NC,), R, 128),
            _row_spec((NC,), R, 128),
            _row_spec((), R, 128),
            _full_spec((128, 512)),
            _full_spec((128, 512)),
            _full_spec((1, 512)),
        ],
        out_specs=[_row_spec((), R, 128)] * 4 + [_row_spec((), R, 16)],
        out_shape=[jax.ShapeDtypeStruct((N_NODES, 128), jnp.float32)] * 4
        + [jax.ShapeDtypeStruct((N_NODES, 16), jnp.float32)],
    )(px, pdeg, x, w1l, w1r, b1r)
    h0, h1_, h2_, h3_, invd = h1c

    ph = _make_sc_agg(4, 128)(src4, dst4, z128, h0, h1_, h2_, h3_)

    w2l = W2_l.reshape(4, 128, 1024)
    w2r = W2_r.reshape(4, 128, 1024)
    b2r = b2[None, :]
    w3lp = jnp.pad(W3_l, ((0, 0), (0, 126))).reshape(8, 128, 128)
    w3rp = jnp.pad(W3_r, ((0, 0), (0, 14))).reshape(8, 128, 16)
    b3p = jnp.pad(b3, (0, 14))[None, :]
    p, pr = pl.pallas_call(
        _l2_body,
        grid=(G,),
        in_specs=[_row_spec((NC,), R, 128)] * 4
        + [_row_spec((), R, 128)] * 4
        + [
            _row_spec((), R, 16),
            _full_spec((4, 128, 1024)),
            _full_spec((4, 128, 1024)),
            _full_spec((1, 1024)),
            _full_spec((8, 128, 128)),
            _full_spec((8, 128, 16)),
            _full_spec((1, 16)),
        ],
        out_specs=[_row_spec((), R, 128), _row_spec((), R, 16)],
        out_shape=[jax.ShapeDtypeStruct((N_NODES, 128), jnp.float32),
                   jax.ShapeDtypeStruct((N_NODES, 16), jnp.float32)],
    )(*ph, h0, h1_, h2_, h3_, invd, w2l, w2r, b2r, w3lp, w3rp, b3p)

    (pp,) = _make_sc_agg(1, 128)(src4, dst4, z128, p)

    out16 = pl.pallas_call(
        _l3_body,
        grid=(G,),
        in_specs=[
            _row_spec((NC,), R, 128),
            _row_spec((), R, 16),
            _row_spec((), R, 16),
        ],
        out_specs=_row_spec((), R, 16),
        out_shape=jax.ShapeDtypeStruct((N_NODES, 16), jnp.float32),
    )(pp, pr, invd)

    return out16[:, :2]


# speedup vs baseline: 2.9068x; 2.9068x over previous
import functools

import jax
import jax.numpy as jnp
from jax import lax
from jax.experimental import pallas as pl
from jax.experimental.pallas import tpu as pltpu
from jax.experimental.pallas import tpu_sc as plsc

N_NODES = 10000
N_EDGES = 160000
NP = 10240
DUMMY = N_NODES
NC, NS = 2, 16
NW = NC * NS
EB = 128
EBD = 128
NBD = 40
E_PAD = EBD * NBD * NW
RPS = NP // NS
NB0, NB1 = 40, 40
NBUF = 2
EPG = NC * NS * NB0 * EB


_MESH = plsc.VectorSubcoreMesh(core_axis_name="c", subcore_axis_name="s")


def _make_sc_agg(n_chunks, dc):
    n = n_chunks

    def body(*refs):
        src4, dst4, zeros = refs[:3]
        tables = refs[3:3 + n]
        outs = refs[3 + n:3 + n + n]
        srcv, dstv = refs[3 + n + n:5 + n + n]
        gbufs = refs[5 + n + n:5 + n + n + NBUF]
        acc = refs[5 + n + n + NBUF]
        sems = refs[6 + n + n + NBUF:6 + n + n + 2 * NBUF]

        c = lax.axis_index("c")
        s = lax.axis_index("s")
        w = c * NS + s
        nb = lax.select(c == 0, NB0, NB1)

        pltpu.sync_copy(src4.at[w], srcv)
        pltpu.sync_copy(dst4.at[w], dstv)

        for i in range(n):
            pltpu.sync_copy(zeros.at[pl.ds(s * RPS, RPS)], acc.at[pl.ds(s * RPS, RPS)])
            plsc.subcore_barrier()

            tbl = tables[i]
            for b in range(NBUF - 1):
                pltpu.async_copy(tbl.at[srcv.at[b]], gbufs[b], sems[b])

            def edge_body(i4, carry):
                for b in range(NBUF):
                    k = i4 * NBUF + b

                    @pl.when(k < nb)
                    def _():
                        pltpu.make_async_copy(tbl.at[srcv.at[k]],
                                              gbufs[b], sems[b]).wait()
                        pltpu.sync_copy(gbufs[b], acc.at[dstv.at[k]], add=True)

                        @pl.when(k + NBUF - 1 < nb)
                        def _():
                            kn = k + NBUF - 1
                            bn = (b + NBUF - 1) % NBUF
                            pltpu.async_copy(tbl.at[srcv.at[kn]],
                                             gbufs[bn], sems[bn])

                return carry

            lax.fori_loop(0, NB0 // NBUF, edge_body, 0)
            plsc.subcore_barrier()
            pltpu.sync_copy(acc.at[pl.ds(s * RPS, RPS)],
                            outs[i].at[c, pl.ds(s * RPS, RPS)])

    return pl.kernel(
        body,
        out_type=[jax.ShapeDtypeStruct((NC, NP, dc), jnp.float32)] * n,
        mesh=_MESH,
        scratch_types=[
            pltpu.VMEM((NB0, EB), jnp.int32),
            pltpu.VMEM((NB0, EB), jnp.int32),
            *[pltpu.VMEM((EB, dc), jnp.float32) for _ in range(NBUF)],
            pltpu.VMEM_SHARED((NP, dc), jnp.float32),
            *[pltpu.SemaphoreType.DMA for _ in range(NBUF)],
        ],
    )


def _sc_deg(dst3, z128, ones128):
    def body(dst3, z128, ones128, out, dstv, onesv, acc):
        c = lax.axis_index("c")
        s = lax.axis_index("s")
        w = c * NS + s
        pltpu.sync_copy(dst3.at[w], dstv)
        pltpu.sync_copy(ones128, onesv)
        pltpu.sync_copy(z128.at[pl.ds(s * RPS, RPS)], acc.at[pl.ds(s * RPS, RPS)])
        plsc.subcore_barrier()

        def edge_body(k, carry):
            pltpu.sync_copy(onesv, acc.at[dstv.at[k]], add=True)
            return carry

        lax.fori_loop(0, NBD, edge_body, 0)
        plsc.subcore_barrier()
        pltpu.sync_copy(acc.at[pl.ds(s * RPS, RPS)], out.at[c, pl.ds(s * RPS, RPS)])

    return pl.kernel(
        body,
        out_type=jax.ShapeDtypeStruct((NC, NP, 128), jnp.float32),
        mesh=_MESH,
        scratch_types=[
            pltpu.VMEM((NBD, EBD), jnp.int32),
            pltpu.VMEM((EBD, 128), jnp.float32),
            pltpu.VMEM_SHARED((NP, 128), jnp.float32),
        ],
    )(dst3, z128, ones128)


R = 400
G = N_NODES // R


def _l1_body(px, pd, x, w1l, w1r, b1, o0, o1, o2, o3, invd):
    deg = (pd[0] + pd[1])[:, 0:16]
    inv = 1.0 / jnp.maximum(deg, 1.0)
    agg = (px[0] + px[1]) * inv[:, 0:1]
    z = (jnp.dot(agg, w1l[...], preferred_element_type=jnp.float32)
         + jnp.dot(x[...], w1r[...], preferred_element_type=jnp.float32)
         + b1[0][None, :])
    h = jnp.maximum(z, 0.0)
    o0[...] = h[:, 0:128]
    o1[...] = h[:, 128:256]
    o2[...] = h[:, 256:384]
    o3[...] = h[:, 384:512]
    invd[...] = inv


def _l2_body(p0, p1, p2, p3, h0, h1, h2c, h3, invd, w2l, w2r, b2, w3lp, w3rp, b3p,
             p_out, pr_out):
    inv1 = invd[...][:, 0:1]
    hs = (h0, h1, h2c, h3)
    ps = (p0, p1, p2, p3)
    z = b2[0][None, :]
    for c in range(4):
        aggc = (ps[c][0] + ps[c][1]) * inv1
        z = z + jnp.dot(aggc, w2l[c], preferred_element_type=jnp.float32)
        z = z + jnp.dot(hs[c][...], w2r[c], preferred_element_type=jnp.float32)
    h2 = jnp.maximum(z, 0.0)
    p = jnp.zeros((R, 128), jnp.float32)
    pr = b3p[0][None, :]
    for c in range(8):
        blk = h2[:, c * 128:(c + 1) * 128]
        p = p + jnp.dot(blk, w3lp[c], preferred_element_type=jnp.float32)
        pr = pr + jnp.dot(blk, w3rp[c], preferred_element_type=jnp.float32)
    p_out[...] = p
    pr_out[...] = pr


def _l3_body(pp, pr, invd, out):
    agg = (pp[0] + pp[1])[:, 0:16] * invd[...]
    out[...] = jnp.maximum(agg + pr[...], 0.0)


def _row_spec(shape_prefix, r, dc):
    nd = len(shape_prefix)
    return pl.BlockSpec(shape_prefix + (r, dc),
                        lambda i, nd=nd: (0,) * nd + (i, 0))


def _full_spec(shape):
    return pl.BlockSpec(shape, lambda i: (0,) * len(shape))


def kernel(x, edge_index, batch, W1_l, W1_r, b1, W2_l, W2_r, b2, W3_l, W3_r, b3):
    src = edge_index[0]
    dst = edge_index[1]
    dst3 = jnp.concatenate(
        [dst, jnp.full((E_PAD - N_EDGES,), DUMMY, jnp.int32)]).reshape(NW, NBD, EBD)

    def _skew(e, fill):
        ep = jnp.concatenate([e, jnp.full((EPG - N_EDGES,), fill, jnp.int32)])
        n0 = NS * NB0 * EB
        e0 = ep[:n0].reshape(NS, NB0, EB)
        e1 = ep[n0:].reshape(NS, NB1, EB)
        e1 = jnp.pad(e1, ((0, 0), (0, NB0 - NB1), (0, 0)))
        return jnp.concatenate([e0, e1])

    src4 = _skew(src, 0)
    dst4 = _skew(dst, DUMMY)
    z128 = jnp.zeros((NP, 128), jnp.float32)
    ones128 = jnp.ones((EBD, 128), jnp.float32)

    (px,) = _make_sc_agg(1, 128)(src4, dst4, z128, x)
    pdeg = _sc_deg(dst3, z128, ones128)

    w1l = W1_l
    w1r = W1_r
    b1r = b1[None, :]
    h1c = pl.pallas_call(
        _l1_body,
        grid=(G,),
        in_specs=[
            _row_spec((NC,), R, 128),
            _row_spec((NC,), R, 128),
            _row_spec((), R, 128),
            _full_spec((128, 512)),
            _full_spec((128, 512)),
            _full_spec((1, 512)),
        ],
        out_specs=[_row_spec((), R, 128)] * 4 + [_row_spec((), R, 16)],
        out_shape=[jax.ShapeDtypeStruct((N_NODES, 128), jnp.float32)] * 4
        + [jax.ShapeDtypeStruct((N_NODES, 16), jnp.float32)],
    )(px, pdeg, x, w1l, w1r, b1r)
    h0, h1_, h2_, h3_, invd = h1c

    ph = _make_sc_agg(4, 128)(src4, dst4, z128, h0, h1_, h2_, h3_)

    w2l = W2_l.reshape(4, 128, 1024)
    w2r = W2_r.reshape(4, 128, 1024)
    b2r = b2[None, :]
    w3lp = jnp.pad(W3_l, ((0, 0), (0, 126))).reshape(8, 128, 128)
    w3rp = jnp.pad(W3_r, ((0, 0), (0, 14))).reshape(8, 128, 16)
    b3p = jnp.pad(b3, (0, 14))[None, :]
    p, pr = pl.pallas_call(
        _l2_body,
        grid=(G,),
        in_specs=[_row_spec((NC,), R, 128)] * 4
        + [_row_spec((), R, 128)] * 4
        + [
            _row_spec((), R, 16),
            _full_spec((4, 128, 1024)),
            _full_spec((4, 128, 1024)),
            _full_spec((1, 1024)),
            _full_spec((8, 128, 128)),
            _full_spec((8, 128, 16)),
            _full_spec((1, 16)),
        ],
        out_specs=[_row_spec((), R, 128), _row_spec((), R, 16)],
        out_shape=[jax.ShapeDtypeStruct((N_NODES, 128), jnp.float32),
                   jax.ShapeDtypeStruct((N_NODES, 16), jnp.float32)],
    )(*ph, h0, h1_, h2_, h3_, invd, w2l, w2r, b2r, w3lp, w3rp, b3p)

    (pp,) = _make_sc_agg(1, 128)(src4, dst4, z128, p)

    out16 = pl.pallas_call(
        _l3_body,
        grid=(G,),
        in_specs=[
            _row_spec((NC,), R, 128),
            _row_spec((), R, 16),
            _row_spec((), R, 16),
        ],
        out_specs=_row_spec((), R, 16),
        out_shape=jax.ShapeDtypeStruct((N_NODES, 16), jnp.float32),
    )(pp, pr, invd)

    return out16[:, :2]
